# baseline (speedup 1.0000x reference)
import functools
import jax
import jax.numpy as jnp
from jax import lax
from jax.experimental import pallas as pl
from jax.experimental.pallas import tpu as pltpu
from jax.experimental.pallas import tpu_sc as plsc

EMBED_DIM = 64
RANK = 16
LANES = 16
FROW = 128
NUM_CORES = 2
NUM_SUBCORES = 16
NUM_WORKERS = NUM_CORES * NUM_SUBCORES
BRPC = 4
TC_B8 = 1000


def _build_fused_table(table128, A128, BD):
    half = table128.shape[0]

    def body(t_ref, a_ref, bd_ref, f_ref):
        a = a_ref[...]
        cs = [
            jnp.dot(a, bd_ref[k], preferred_element_type=jnp.float32)
            for k in range(4)
        ]
        c = jnp.stack(cs, axis=1)
        c2 = c.reshape(4 * TC_B8, FROW)
        f_ref[...] = t_ref[...] + c2

    return pl.pallas_call(
        body,
        grid=(half // (4 * TC_B8),),
        in_specs=[
            pl.BlockSpec((4 * TC_B8, FROW), lambda i: (i, 0)),
            pl.BlockSpec((TC_B8, FROW), lambda i: (i, 0)),
            pl.BlockSpec((4, FROW, FROW), lambda i: (0, 0, 0)),
        ],
        out_specs=pl.BlockSpec((4 * TC_B8, FROW), lambda i: (i, 0)),
        out_shape=jax.ShapeDtypeStruct((half, FROW), jnp.float32),
    )(table128, A128, BD)


def _sc_lookup(idx, F2, batch, hist):
    br_per_worker = batch // NUM_WORKERS
    num_chunks = br_per_worker // BRPC
    mesh = plsc.VectorSubcoreMesh(core_axis_name="c", subcore_axis_name="s")
    win = [0, 16, 32, 34]

    @functools.partial(
        pl.kernel,
        mesh=mesh,
        compiler_params=pltpu.CompilerParams(use_tc_tiling_on_sc=True),
        out_type=jax.ShapeDtypeStruct((batch, hist, EMBED_DIM), jnp.float32),
        scratch_types=[
            pltpu.VMEM((2, BRPC, hist), jnp.int32),
            pltpu.VMEM((2, BRPC, hist), jnp.int32),
            pltpu.VMEM((2, BRPC, hist, FROW), jnp.float32),
            pltpu.VMEM((2, BRPC, hist, EMBED_DIM), jnp.float32),
            pltpu.SemaphoreType.DMA((2,)),
            pltpu.SemaphoreType.DMA((2,)),
            pltpu.SemaphoreType.DMA((2,)),
        ],
    )
    def kern(idx_hbm, f_hbm, out_hbm, slab_v, pslab_v, rows_v, obuf_v,
             isem, gsem, osem):
        wid = lax.axis_index("s") * NUM_CORES + lax.axis_index("c")
        br0 = wid * br_per_worker

        def slab_copy(c, b):
            return pltpu.make_async_copy(
                idx_hbm.at[pl.ds(br0 + c * BRPC, BRPC)], slab_v.at[b],
                isem.at[b])

        def gather(b, r):
            return pltpu.make_async_copy(
                f_hbm.at[pslab_v.at[b, r]], rows_v.at[b, r], gsem.at[b])

        def out_copy(c, b):
            return pltpu.make_async_copy(
                obuf_v.at[b], out_hbm.at[pl.ds(br0 + c * BRPC, BRPC)],
                osem.at[b])

        slab_copy(0, 0).start()

        def chunk_body(c, _):
            b = lax.rem(c, 2)
            slab_copy(c, b).wait()

            for r in range(BRPC):
                for k0 in win:
                    pslab_v[b, r, pl.ds(k0, LANES)] = (
                        lax.shift_right_logical(
                            slab_v[b, r, pl.ds(k0, LANES)], 1))
            for r in range(BRPC):
                gather(b, r).start()

            @pl.when(c + 1 < num_chunks)
            def _():
                slab_copy(c + 1, 1 - b).start()

            for r in range(BRPC):
                gather(b, r).wait()

            @pl.when(c >= 2)
            def _():
                out_copy(c - 2, b).wait()

            for r in range(BRPC):
                offs = [
                    lax.bitwise_and(slab_v[b, r, pl.ds(k0, LANES)], 1)
                    * EMBED_DIM
                    for k0 in win
                ]
                for l in range(hist):
                    vi, li = (3, l - 34) if l >= 48 else (l // 16, l % 16)
                    h = offs[vi][li]
                    for k in range(EMBED_DIM // LANES):
                        obuf_v[b, r, l, pl.ds(k * LANES, LANES)] = (
                            rows_v[b, r, l, pl.ds(h + k * LANES, LANES)])

            out_copy(c, b).start()
            return 0

        lax.fori_loop(0, num_chunks, chunk_body, 0)
        out_copy(num_chunks - 2, 0).wait()
        out_copy(num_chunks - 1, 1).wait()

    return kern(idx, F2)


def kernel(input, table, A, B_w, C_w):
    B, L = input.shape
    V = table.shape[0]
    M = B_w.T @ C_w.T
    BD = jnp.zeros((4, FROW, FROW), dtype=jnp.float32)
    for k in range(4):
        BD = BD.at[k, 32 * k:32 * k + RANK, 0:EMBED_DIM].set(M)
        BD = BD.at[k, 32 * k + RANK:32 * k + 2 * RANK,
                   EMBED_DIM:2 * EMBED_DIM].set(M)
    table128 = jnp.reshape(table, (V // 2, FROW))
    A128 = jnp.reshape(A, (V // 8, FROW))
    F2 = _build_fused_table(table128, A128, BD)
    return _sc_lookup(input.astype(jnp.int32), F2, B, L)
